# baseline (speedup 1.0000x reference)
import functools

import jax
import jax.numpy as jnp
from jax import lax
from jax.experimental import pallas as pl
from jax.experimental.pallas import tpu as pltpu
from jax.experimental.pallas import tpu_sc as plsc

_BATCH = 16384
_K = 64
_NUM_WORKERS = 32
_BPW = _BATCH // _NUM_WORKERS
_CHUNK = 256
_NCHUNKS = _BPW // _CHUNK


def _nmf_body(u_hbm, i_hbm, r_hbm, out_hbm,
              idx_u, idx_i, rows_u, rows_i, out_v, sem_u, sem_i):
    cid = lax.axis_index("c")
    sid = lax.axis_index("s")
    wid = sid * 2 + cid
    base = pl.multiple_of(wid * _BPW, _BPW)

    pltpu.sync_copy(u_hbm.at[pl.ds(base, _BPW)], idx_u)
    pltpu.sync_copy(i_hbm.at[pl.ds(base, _BPW)], idx_i)

    lanes = lax.iota(jnp.int32, 16)

    for chunk in range(_NCHUNKS):
        off = chunk * _CHUNK
        cu = pltpu.async_copy(
            r_hbm.at[idx_u.at[pl.ds(off, _CHUNK)]], rows_u, sem_u)
        ci = pltpu.async_copy(
            r_hbm.at[idx_i.at[pl.ds(off, _CHUNK)]], rows_i, sem_i)
        cu.wait()
        ci.wait()

        def grp(g, carry, _off=off):
            row = g * 16 + lanes
            acc = jnp.zeros((16,), jnp.float32)
            for k in range(_K):
                col = (lanes + k) & (_K - 1)
                acc = acc + (plsc.load_gather(rows_u, [row, col])
                             * plsc.load_gather(rows_i, [row, col + _K]))
            out_v[pl.ds(pl.multiple_of(_off + g * 16, 16), 16)] = acc
            return carry

        lax.fori_loop(0, _CHUNK // 16, grp, 0)

    pltpu.sync_copy(out_v, out_hbm.at[pl.ds(base, _BPW)])


_nmf_sc = functools.partial(
    pl.kernel,
    out_type=jax.ShapeDtypeStruct((_BATCH,), jnp.float32),
    mesh=plsc.VectorSubcoreMesh(core_axis_name="c", subcore_axis_name="s"),
    compiler_params=pltpu.CompilerParams(needs_layout_passes=False),
    scratch_types=[
        pltpu.VMEM((_BPW,), jnp.int32),
        pltpu.VMEM((_BPW,), jnp.int32),
        pltpu.VMEM((_CHUNK, 2 * _K), jnp.float32),
        pltpu.VMEM((_CHUNK, 2 * _K), jnp.float32),
        pltpu.VMEM((_BPW,), jnp.float32),
        pltpu.SemaphoreType.DMA,
        pltpu.SemaphoreType.DMA,
    ],
)(_nmf_body)


_CROWS = 4000


def _concat_body(p_ref, q_ref, o_ref):
    o_ref[...] = jnp.concatenate([p_ref[...], q_ref[...]], axis=1)


_concat_tc = pl.pallas_call(
    _concat_body,
    grid=(100000 // _CROWS,),
    in_specs=[
        pl.BlockSpec((_CROWS, _K), lambda i: (i, 0)),
        pl.BlockSpec((_CROWS, _K), lambda i: (i, 0)),
    ],
    out_specs=pl.BlockSpec((_CROWS, 2 * _K), lambda i: (i, 0)),
    out_shape=jax.ShapeDtypeStruct((100000, 2 * _K), jnp.float32),
)


def kernel(train_x, P, Q):
    user_id = train_x[:, 0].astype(jnp.int32)
    item_id = train_x[:, 1].astype(jnp.int32)
    R = _concat_tc(P, Q)
    return _nmf_sc(user_id, item_id, R)


# speedup vs baseline: 1.4927x; 1.4927x over previous
import functools

import jax
import jax.numpy as jnp
from jax import lax
from jax.experimental import pallas as pl
from jax.experimental.pallas import tpu as pltpu
from jax.experimental.pallas import tpu_sc as plsc

_BATCH = 16384
_K = 64
_NUM_WORKERS = 32
_BPW = _BATCH // _NUM_WORKERS
_CHUNK = 256
_NCHUNKS = _BPW // _CHUNK


def _nmf_body(u_hbm, i_hbm, p_hbm, q_hbm, out_hbm,
              idx_u, idx_i, rows_u, rows_i, out_v, sem_u, sem_i):
    cid = lax.axis_index("c")
    sid = lax.axis_index("s")
    wid = sid * 2 + cid
    base = pl.multiple_of(wid * _BPW, _BPW)

    pltpu.sync_copy(u_hbm.at[pl.ds(base, _BPW)], idx_u)
    pltpu.sync_copy(i_hbm.at[pl.ds(base, _BPW)], idx_i)

    lanes = lax.iota(jnp.int32, 16)

    for chunk in range(_NCHUNKS):
        off = chunk * _CHUNK

        def fire(g, carry, _off=off):
            s = pl.multiple_of(_off + g * 16, 16)
            vu = idx_u[pl.ds(s, 16)]
            vi = idx_i[pl.ds(s, 16)]
            for j in range(16):
                ru = jnp.sum(jnp.where(lanes == j, vu, 0))
                ri = jnp.sum(jnp.where(lanes == j, vi, 0))
                b = g * 16 + j
                pltpu.make_async_copy(
                    p_hbm.at[ru], rows_u.at[b], sem_u).start()
                pltpu.make_async_copy(
                    q_hbm.at[ri], rows_i.at[b], sem_i).start()
            return carry

        lax.fori_loop(0, _CHUNK // 16, fire, 0)

        def drain(b, carry):
            pltpu.make_async_copy(
                p_hbm.at[0], rows_u.at[0], sem_u).wait()
            pltpu.make_async_copy(
                q_hbm.at[0], rows_i.at[0], sem_i).wait()
            return carry

        lax.fori_loop(0, _CHUNK, drain, 0)

        def grp(g, carry, _off=off):
            row = g * 16 + lanes
            acc = jnp.zeros((16,), jnp.float32)
            for k in range(_K):
                col = (lanes + k) & (_K - 1)
                acc = acc + (plsc.load_gather(rows_u, [row, col])
                             * plsc.load_gather(rows_i, [row, col]))
            out_v[pl.ds(pl.multiple_of(_off + g * 16, 16), 16)] = acc
            return carry

        lax.fori_loop(0, _CHUNK // 16, grp, 0)

    pltpu.sync_copy(out_v, out_hbm.at[pl.ds(base, _BPW)])


_nmf_sc = functools.partial(
    pl.kernel,
    out_type=jax.ShapeDtypeStruct((_BATCH,), jnp.float32),
    mesh=plsc.VectorSubcoreMesh(core_axis_name="c", subcore_axis_name="s"),
    compiler_params=pltpu.CompilerParams(needs_layout_passes=False),
    scratch_types=[
        pltpu.VMEM((_BPW,), jnp.int32),
        pltpu.VMEM((_BPW,), jnp.int32),
        pltpu.VMEM((_CHUNK, _K), jnp.float32),
        pltpu.VMEM((_CHUNK, _K), jnp.float32),
        pltpu.VMEM((_BPW,), jnp.float32),
        pltpu.SemaphoreType.DMA,
        pltpu.SemaphoreType.DMA,
    ],
)(_nmf_body)


def kernel(train_x, P, Q):
    user_id = train_x[:, 0].astype(jnp.int32)
    item_id = train_x[:, 1].astype(jnp.int32)
    return _nmf_sc(user_id, item_id, P, Q)


# speedup vs baseline: 1.4953x; 1.0017x over previous
import functools

import jax
import jax.numpy as jnp
from jax import lax
from jax.experimental import pallas as pl
from jax.experimental.pallas import tpu as pltpu
from jax.experimental.pallas import tpu_sc as plsc

_BATCH = 16384
_K = 64
_NUM_WORKERS = 32
_BPW = _BATCH // _NUM_WORKERS
_CHUNK = 256
_NCHUNKS = _BPW // _CHUNK


def _nmf_body(u_hbm, i_hbm, p_hbm, q_hbm, out_hbm,
              idx_u, idx_i, rows_u, rows_i, out_v, sem_u, sem_i):
    cid = lax.axis_index("c")
    sid = lax.axis_index("s")
    wid = sid * 2 + cid
    base = pl.multiple_of(wid * _BPW, _BPW)

    pltpu.sync_copy(u_hbm.at[pl.ds(base, _BPW)], idx_u)
    pltpu.sync_copy(i_hbm.at[pl.ds(base, _BPW)], idx_i)

    lanes = lax.iota(jnp.int32, 16)

    for chunk in range(_NCHUNKS):
        off = chunk * _CHUNK

        def fire(g, carry, _off=off):
            s = pl.multiple_of(_off + g * 16, 16)
            vu = idx_u[pl.ds(s, 16)]
            vi = idx_i[pl.ds(s, 16)]
            for j in range(16):
                ru = jnp.sum(jnp.where(lanes == j, vu, 0))
                ri = jnp.sum(jnp.where(lanes == j, vi, 0))
                b = g * 16 + j
                pltpu.make_async_copy(
                    p_hbm.at[ru], rows_u.at[b], sem_u).start()
                pltpu.make_async_copy(
                    q_hbm.at[ri], rows_i.at[b], sem_i).start()
            return carry

        lax.fori_loop(0, _CHUNK // 16, fire, 0)

        def drain(b, carry):
            pltpu.make_async_copy(
                p_hbm.at[0], rows_u.at[0], sem_u).wait()
            pltpu.make_async_copy(
                q_hbm.at[0], rows_i.at[0], sem_i).wait()
            return carry

        lax.fori_loop(0, _CHUNK, drain, 0)

        def grp(g, carry, _off=off):
            row = g * 16 + lanes
            acc = jnp.zeros((16,), jnp.float32)
            for k in range(_K):
                col = (lanes + k) & (_K - 1)
                acc = acc + (plsc.load_gather(rows_u, [row, col])
                             * plsc.load_gather(rows_i, [row, col]))
            out_v[pl.ds(pl.multiple_of(_off + g * 16, 16), 16)] = acc
            return carry

        lax.fori_loop(0, _CHUNK // 16, grp, 0)

    pltpu.sync_copy(out_v, out_hbm.at[pl.ds(base, _BPW)])


_nmf_sc = functools.partial(
    pl.kernel,
    out_type=jax.ShapeDtypeStruct((_BATCH,), jnp.float32),
    mesh=plsc.VectorSubcoreMesh(core_axis_name="c", subcore_axis_name="s"),
    compiler_params=pltpu.CompilerParams(needs_layout_passes=False),
    scratch_types=[
        pltpu.VMEM((_BPW,), jnp.int32),
        pltpu.VMEM((_BPW,), jnp.int32),
        pltpu.VMEM((_CHUNK, _K), jnp.float32),
        pltpu.VMEM((_CHUNK, _K), jnp.float32),
        pltpu.VMEM((_BPW,), jnp.float32),
        pltpu.SemaphoreType.DMA,
        pltpu.SemaphoreType.DMA,
    ],
)(_nmf_body)


def kernel(train_x, P, Q):
    user_id = train_x[:, 0].astype(jnp.int32)
    item_id = train_x[:, 1].astype(jnp.int32)
    return _nmf_sc(user_id, item_id, jnp.copy(P), jnp.copy(Q))
